# baseline (speedup 1.0000x reference)
import jax
import jax.numpy as jnp
from jax.experimental import pallas as pl
from jax.experimental.pallas import tpu as pltpu

_NUM_HOP = 4
_TH = 0.6
_T = 0.07
_NC = 128
_N = 4096
_IN = 256
_TD = 512
_NTD = _N // _TD
_TILE = 512
_NT = _N // _TILE


def _nt_dot(a, b):
    return jax.lax.dot_general(a, b, (((1,), (1,)), ((), ())),
                               preferred_element_type=jnp.float32)


def _nn_dot(a, b):
    return jax.lax.dot_general(a, b, (((1,), (0,)), ((), ())),
                               preferred_element_type=jnp.float32)


def _l2norm_rows(x):
    return x / jnp.sqrt(jnp.sum(x * x, axis=1, keepdims=True))


def _gnn_body(feat_s_ref, feat_t_ref, We_ref, be_ref, Wt_ref, bt_ref,
              out_ref,
              F8_ref, Gbf_ref, H_ref, stack_ref, acc_t_ref, acc_s_ref,
              norm_ref, fgt_ref, fgs_ref, L0_ref, L1_ref,
              cnt_ref, part_ref):
    bf16 = jnp.bfloat16
    fp8 = jnp.float8_e4m3fn
    We = We_ref[...].astype(bf16)
    be = be_ref[...]

    ft = _l2norm_rows(_nn_dot(feat_t_ref[...].astype(bf16), We) + be)
    fs = _l2norm_rows(_nn_dot(feat_s_ref[...].astype(bf16), We) + be)
    H_ref[:, :_NC] = ft
    H_ref[:, _NC:] = fs
    F8_ref[...] = ft.astype(fp8)
    stack_ref[:_N, :_NC] = ft.astype(bf16)
    stack_ref[_N:, :_NC] = fs.astype(bf16)

    def deg_mm(t, dst):
        rows = pl.ds(t * _TD, _TD)
        dst[...] = _nt_dot(F8_ref[rows, :], F8_ref[...])

    def deg_proc(t, src):
        rows = pl.ds(t * _TD, _TD)
        m = (src[...] > _TH).astype(jnp.float32)
        deg = jnp.sum(m, axis=1, keepdims=True) + 1.0
        norm_ref[rows, :] = jax.lax.rsqrt(jnp.maximum(deg, 1.0))
        cnt_ref[t] = jnp.sum(deg) - 2.0 * float(_TD)

    deg_mm(0, L0_ref)

    def deg_step(i, carry):
        @pl.when(jax.lax.rem(i, 2) == 0)
        def _():
            deg_mm(i + 1, L1_ref)
            deg_proc(i, L0_ref)

        @pl.when(jax.lax.rem(i, 2) == 1)
        def _():
            deg_mm(i + 1, L0_ref)
            deg_proc(i, L1_ref)

        return carry

    jax.lax.fori_loop(0, _NTD - 1, deg_step, 0)
    deg_proc(_NTD - 1, L1_ref)

    for hop in range(_NUM_HOP):
        Gbf_ref[...] = (H_ref[...] * norm_ref[...]).astype(bf16)

        def hop_tile(t, carry):
            rows = pl.ds(t * _TD, _TD)

            @pl.when(cnt_ref[t] > 0.5)
            def _():
                d = _nt_dot(F8_ref[rows, :], F8_ref[...])
                m = (d > _TH).astype(bf16)
                s = _nn_dot(m, Gbf_ref[...])
                nrm = norm_ref[rows, :]
                H_ref[rows, :] = nrm * s + (nrm * nrm) * H_ref[rows, :]

            return carry

        jax.lax.fori_loop(0, _NTD, hop_tile, 0)
        Hb = H_ref[...].astype(bf16)
        c = (hop + 1) * _NC
        stack_ref[:_N, c:c + _NC] = Hb[:, :_NC]
        stack_ref[_N:, c:c + _NC] = Hb[:, _NC:]

    Wt = Wt_ref[...].astype(bf16)
    bt = bt_ref[...]
    gt = _l2norm_rows(_nn_dot(stack_ref[:_N, :], Wt) + bt)
    gs = _l2norm_rows(_nn_dot(stack_ref[_N:, :], Wt) + bt)
    acc_t_ref[...] = gt
    acc_s_ref[...] = gs
    inv_t = 1.0 / _T
    fgt_ref[...] = (gt * inv_t).astype(fp8)
    fgs_ref[...] = gs.astype(fp8)

    def nce_mm(t, dst):
        rows = pl.ds(t * _TILE, _TILE)
        dst[...] = _nt_dot(fgt_ref[rows, :], fgs_ref[...])

    def nce_proc(t, src):
        rows = pl.ds(t * _TILE, _TILE)
        col = jax.lax.broadcasted_iota(jnp.int32, (_TILE, _N), 1)
        row = jax.lax.broadcasted_iota(jnp.int32, (_TILE, _N), 0) + t * _TILE
        logits = jnp.where(col == row, -10.0 * inv_t, src[...])
        pos = jnp.sum(acc_t_ref[rows, :] * acc_s_ref[rows, :], axis=1,
                      keepdims=True) * inv_t
        se = jnp.sum(jnp.exp(logits), axis=1, keepdims=True) + jnp.exp(pos)
        part_ref[t] = jnp.sum(jnp.log(se) - pos)

    nce_mm(0, L0_ref)

    def nce_step(i, carry):
        @pl.when(jax.lax.rem(i, 2) == 0)
        def _():
            nce_mm(i + 1, L1_ref)
            nce_proc(i, L0_ref)

        @pl.when(jax.lax.rem(i, 2) == 1)
        def _():
            nce_mm(i + 1, L0_ref)
            nce_proc(i, L1_ref)

        return carry

    jax.lax.fori_loop(0, _NT - 1, nce_step, 0)
    nce_proc(_NT - 1, L1_ref)

    total = part_ref[0]
    for t in range(1, _NT):
        total = total + part_ref[t]
    out_ref[0, 0] = total * (1.0 / float(_N))


def kernel(feat_s, feat_t, W_embed, b_embed, W_tag, b_tag):
    res = pl.pallas_call(
        _gnn_body,
        out_shape=jax.ShapeDtypeStruct((1, 1), jnp.float32),
        in_specs=[pl.BlockSpec(memory_space=pltpu.VMEM)] * 6,
        out_specs=pl.BlockSpec(memory_space=pltpu.SMEM),
        scratch_shapes=[
            pltpu.VMEM((_N, _NC), jnp.float8_e4m3fn),
            pltpu.VMEM((_N, 2 * _NC), jnp.bfloat16),
            pltpu.VMEM((_N, 2 * _NC), jnp.float32),
            pltpu.VMEM((2 * _N, 5 * _NC), jnp.bfloat16),
            pltpu.VMEM((_N, _NC), jnp.float32),
            pltpu.VMEM((_N, _NC), jnp.float32),
            pltpu.VMEM((_N, 1), jnp.float32),
            pltpu.VMEM((_N, _NC), jnp.float8_e4m3fn),
            pltpu.VMEM((_N, _NC), jnp.float8_e4m3fn),
            pltpu.VMEM((_TILE, _N), jnp.float32),
            pltpu.VMEM((_TILE, _N), jnp.float32),
            pltpu.SMEM((_NTD,), jnp.float32),
            pltpu.SMEM((_NT,), jnp.float32),
        ],
    )(feat_s, feat_t, W_embed, b_embed.reshape(1, _NC),
      W_tag, b_tag.reshape(1, _NC))
    return res[0, 0]


# speedup vs baseline: 1.0402x; 1.0402x over previous
import jax
import jax.numpy as jnp
from jax.experimental import pallas as pl
from jax.experimental.pallas import tpu as pltpu

_NUM_HOP = 4
_TH = 0.6
_T = 0.07
_NC = 128
_N = 4096
_IN = 256
_TD = 1024
_NTD = _N // _TD
_TILE = 512
_NT = _N // _TILE


def _nt_dot(a, b):
    return jax.lax.dot_general(a, b, (((1,), (1,)), ((), ())),
                               preferred_element_type=jnp.float32)


def _nt_dot_bf(a, b):
    return jax.lax.dot_general(a, b, (((1,), (1,)), ((), ())),
                               preferred_element_type=jnp.float32
                               ).astype(jnp.bfloat16)


def _nn_dot(a, b):
    return jax.lax.dot_general(a, b, (((1,), (0,)), ((), ())),
                               preferred_element_type=jnp.float32)


def _l2norm_rows(x):
    return x / jnp.sqrt(jnp.sum(x * x, axis=1, keepdims=True))


def _gnn_body(feat_s_ref, feat_t_ref, We_ref, be_ref, Wt_ref, bt_ref,
              out_ref,
              F8_ref, Gbf_ref, H_ref, stack_ref, acc_t_ref, acc_s_ref,
              norm_ref, fgt_ref, fgs_ref, cnt_ref, part_ref):
    bf16 = jnp.bfloat16
    fp8 = jnp.float8_e4m3fn
    We = We_ref[...].astype(bf16)
    be = be_ref[...]

    ft = _l2norm_rows(_nn_dot(feat_t_ref[...].astype(bf16), We) + be)
    fs = _l2norm_rows(_nn_dot(feat_s_ref[...].astype(bf16), We) + be)
    H_ref[:, :_NC] = ft
    H_ref[:, _NC:] = fs
    F8_ref[...] = ft.astype(fp8)
    stack_ref[:_N, :_NC] = ft.astype(bf16)
    stack_ref[_N:, :_NC] = fs.astype(bf16)

    def deg_tile(t, carry):
        rows = pl.ds(t * _TD, _TD)
        d = _nt_dot_bf(F8_ref[rows, :], F8_ref[...])
        m = d > _TH
        deg = jnp.sum(m, axis=1, keepdims=True,
                      dtype=jnp.float32) + 1.0
        norm_ref[rows, :] = jax.lax.rsqrt(jnp.maximum(deg, 1.0))
        cnt_ref[t] = jnp.sum(deg) - 2.0 * float(_TD)
        return carry

    jax.lax.fori_loop(0, _NTD, deg_tile, 0)

    tot = cnt_ref[0]
    for t in range(1, _NTD):
        tot = tot + cnt_ref[t]
    bt = bt_ref[...]
    inv_t = 1.0 / _T

    @pl.when(tot > 0.5)
    def _():
        for hop in range(_NUM_HOP):
            Gbf_ref[...] = (H_ref[...] * norm_ref[...]).astype(bf16)

            def hop_tile(t, carry):
                rows = pl.ds(t * _TD, _TD)

                @pl.when(cnt_ref[t] > 0.5)
                def _():
                    d = _nt_dot(F8_ref[rows, :], F8_ref[...])
                    m = (d > _TH).astype(bf16)
                    s = _nn_dot(m, Gbf_ref[...])
                    nrm = norm_ref[rows, :]
                    H_ref[rows, :] = nrm * s + (nrm * nrm) * H_ref[rows, :]

                return carry

            jax.lax.fori_loop(0, _NTD, hop_tile, 0)
            Hb = H_ref[...].astype(bf16)
            c = (hop + 1) * _NC
            stack_ref[:_N, c:c + _NC] = Hb[:, :_NC]
            stack_ref[_N:, c:c + _NC] = Hb[:, _NC:]

        Wt = Wt_ref[...].astype(bf16)
        gt = _l2norm_rows(_nn_dot(stack_ref[:_N, :], Wt) + bt)
        gs = _l2norm_rows(_nn_dot(stack_ref[_N:, :], Wt) + bt)
        acc_t_ref[...] = gt
        acc_s_ref[...] = gs
        fgt_ref[...] = (gt * inv_t).astype(fp8)
        fgs_ref[...] = gs.astype(fp8)

    @pl.when(tot <= 0.5)
    def _():
        Wsum = Wt_ref[0 * _NC:1 * _NC, :]
        for k in range(1, _NUM_HOP + 1):
            Wsum = Wsum + Wt_ref[k * _NC:(k + 1) * _NC, :]
        Wsum = Wsum.astype(bf16)
        gt = _l2norm_rows(_nn_dot(stack_ref[:_N, :_NC], Wsum) + bt)
        gs = _l2norm_rows(_nn_dot(stack_ref[_N:, :_NC], Wsum) + bt)
        acc_t_ref[...] = gt
        acc_s_ref[...] = gs
        fgt_ref[...] = (gt * inv_t).astype(fp8)
        fgs_ref[...] = gs.astype(fp8)

    def nce_tile(t, carry):
        rows = pl.ds(t * _TILE, _TILE)
        logits = _nt_dot_bf(fgt_ref[rows, :], fgs_ref[...])
        col = jax.lax.broadcasted_iota(jnp.int32, (_TILE, _N), 1)
        row = jax.lax.broadcasted_iota(jnp.int32, (_TILE, _N), 0) + t * _TILE
        logits = jnp.where(col == row, jnp.bfloat16(-10.0 * inv_t), logits)
        pos = jnp.sum(acc_t_ref[rows, :] * acc_s_ref[rows, :], axis=1,
                      keepdims=True) * inv_t
        se = jnp.sum(jnp.exp(logits), axis=1, keepdims=True,
                     dtype=jnp.float32) + jnp.exp(pos)
        part_ref[t] = jnp.sum(jnp.log(se) - pos)
        return carry

    jax.lax.fori_loop(0, _NT, nce_tile, 0)

    total = part_ref[0]
    for t in range(1, _NT):
        total = total + part_ref[t]
    out_ref[0, 0] = total * (1.0 / float(_N))


def kernel(feat_s, feat_t, W_embed, b_embed, W_tag, b_tag):
    res = pl.pallas_call(
        _gnn_body,
        out_shape=jax.ShapeDtypeStruct((1, 1), jnp.float32),
        in_specs=[pl.BlockSpec(memory_space=pltpu.VMEM)] * 6,
        out_specs=pl.BlockSpec(memory_space=pltpu.SMEM),
        scratch_shapes=[
            pltpu.VMEM((_N, _NC), jnp.float8_e4m3fn),
            pltpu.VMEM((_N, 2 * _NC), jnp.bfloat16),
            pltpu.VMEM((_N, 2 * _NC), jnp.float32),
            pltpu.VMEM((2 * _N, 5 * _NC), jnp.bfloat16),
            pltpu.VMEM((_N, _NC), jnp.float32),
            pltpu.VMEM((_N, _NC), jnp.float32),
            pltpu.VMEM((_N, 1), jnp.float32),
            pltpu.VMEM((_N, _NC), jnp.float8_e4m3fn),
            pltpu.VMEM((_N, _NC), jnp.float8_e4m3fn),
            pltpu.SMEM((_NTD,), jnp.float32),
            pltpu.SMEM((_NT,), jnp.float32),
        ],
    )(feat_s, feat_t, W_embed, b_embed.reshape(1, _NC),
      W_tag, b_tag.reshape(1, _NC))
    return res[0, 0]


# speedup vs baseline: 1.2467x; 1.1985x over previous
import jax
import jax.numpy as jnp
from jax.experimental import pallas as pl
from jax.experimental.pallas import tpu as pltpu

_NUM_HOP = 4
_TH = 0.6
_T = 0.07
_NC = 128
_N = 4096
_IN = 256
_TD = 1024
_NTD = _N // _TD
_TILE = 512
_NT = _N // _TILE


def _nt_dot(a, b):
    return jax.lax.dot_general(a, b, (((1,), (1,)), ((), ())),
                               preferred_element_type=jnp.float32)


def _nt_dot_bf(a, b):
    return jax.lax.dot_general(a, b, (((1,), (1,)), ((), ())),
                               preferred_element_type=jnp.float32
                               ).astype(jnp.bfloat16)


def _nn_dot(a, b):
    return jax.lax.dot_general(a, b, (((1,), (0,)), ((), ())),
                               preferred_element_type=jnp.float32)


def _l2norm_rows(x):
    return x / jnp.sqrt(jnp.sum(x * x, axis=1, keepdims=True))


def _gnn_body(feat_s_ref, feat_t_ref, We_ref, be_ref, Wt_ref, bt_ref,
              out_ref,
              F8_ref, Gbf_ref, H_ref, stack_ref, acc_t_ref, acc_s_ref,
              norm_ref, fgt_ref, fgs_ref, cnt_ref, part_ref):
    bf16 = jnp.bfloat16
    fp8 = jnp.float8_e4m3fn
    We = We_ref[...].astype(bf16)
    be = be_ref[...]

    ft = _l2norm_rows(_nn_dot(feat_t_ref[...].astype(bf16), We) + be)
    fs = _l2norm_rows(_nn_dot(feat_s_ref[...].astype(bf16), We) + be)
    H_ref[:, :_NC] = ft
    H_ref[:, _NC:] = fs
    F8_ref[...] = ft.astype(fp8)
    stack_ref[:_N, :_NC] = ft.astype(bf16)
    stack_ref[_N:, :_NC] = fs.astype(bf16)

    def deg_tile(t, carry):
        rows = pl.ds(t * _TD, _TD)
        d = _nt_dot(F8_ref[rows, :], F8_ref[...])
        m = d > _TH
        deg = jnp.sum(m, axis=1, keepdims=True,
                      dtype=jnp.float32) + 1.0
        norm_ref[rows, :] = jax.lax.rsqrt(jnp.maximum(deg, 1.0))
        cnt_ref[t] = jnp.sum(deg) - 2.0 * float(_TD)
        return carry

    jax.lax.fori_loop(0, _NTD, deg_tile, 0)

    tot = cnt_ref[0]
    for t in range(1, _NTD):
        tot = tot + cnt_ref[t]
    bt = bt_ref[...]
    inv_t = 1.0 / _T

    @pl.when(tot > 0.5)
    def _():
        for hop in range(_NUM_HOP):
            Gbf_ref[...] = (H_ref[...] * norm_ref[...]).astype(bf16)

            def hop_tile(t, carry):
                rows = pl.ds(t * _TD, _TD)

                @pl.when(cnt_ref[t] > 0.5)
                def _():
                    d = _nt_dot(F8_ref[rows, :], F8_ref[...])
                    m = (d > _TH).astype(bf16)
                    s = _nn_dot(m, Gbf_ref[...])
                    nrm = norm_ref[rows, :]
                    H_ref[rows, :] = nrm * s + (nrm * nrm) * H_ref[rows, :]

                return carry

            jax.lax.fori_loop(0, _NTD, hop_tile, 0)
            Hb = H_ref[...].astype(bf16)
            c = (hop + 1) * _NC
            stack_ref[:_N, c:c + _NC] = Hb[:, :_NC]
            stack_ref[_N:, c:c + _NC] = Hb[:, _NC:]

        Wt = Wt_ref[...].astype(bf16)
        gt = _l2norm_rows(_nn_dot(stack_ref[:_N, :], Wt) + bt)
        gs = _l2norm_rows(_nn_dot(stack_ref[_N:, :], Wt) + bt)
        acc_t_ref[...] = gt
        acc_s_ref[...] = gs
        fgt_ref[...] = (gt * inv_t).astype(fp8)
        fgs_ref[...] = gs.astype(fp8)

    @pl.when(tot <= 0.5)
    def _():
        Wsum = Wt_ref[0 * _NC:1 * _NC, :]
        for k in range(1, _NUM_HOP + 1):
            Wsum = Wsum + Wt_ref[k * _NC:(k + 1) * _NC, :]
        Wsum = Wsum.astype(bf16)
        gt = _l2norm_rows(_nn_dot(stack_ref[:_N, :_NC], Wsum) + bt)
        gs = _l2norm_rows(_nn_dot(stack_ref[_N:, :_NC], Wsum) + bt)
        acc_t_ref[...] = gt
        acc_s_ref[...] = gs
        fgt_ref[...] = (gt * inv_t).astype(fp8)
        fgs_ref[...] = gs.astype(fp8)

    def nce_tile(t, carry):
        rows = pl.ds(t * _TILE, _TILE)
        logits = _nt_dot(fgt_ref[rows, :], fgs_ref[...])
        col = jax.lax.broadcasted_iota(jnp.int32, (_TILE, _N), 1)
        row = jax.lax.broadcasted_iota(jnp.int32, (_TILE, _N), 0) + t * _TILE
        logits = jnp.where(col == row, -10.0 * inv_t, logits)
        pos = jnp.sum(acc_t_ref[rows, :] * acc_s_ref[rows, :], axis=1,
                      keepdims=True) * inv_t
        se = jnp.sum(jnp.exp(logits), axis=1, keepdims=True,
                     dtype=jnp.float32) + jnp.exp(pos)
        part_ref[t] = jnp.sum(jnp.log(se) - pos)
        return carry

    jax.lax.fori_loop(0, _NT, nce_tile, 0)

    total = part_ref[0]
    for t in range(1, _NT):
        total = total + part_ref[t]
    out_ref[0, 0] = total * (1.0 / float(_N))


def kernel(feat_s, feat_t, W_embed, b_embed, W_tag, b_tag):
    res = pl.pallas_call(
        _gnn_body,
        out_shape=jax.ShapeDtypeStruct((1, 1), jnp.float32),
        in_specs=[pl.BlockSpec(memory_space=pltpu.VMEM)] * 6,
        out_specs=pl.BlockSpec(memory_space=pltpu.SMEM),
        scratch_shapes=[
            pltpu.VMEM((_N, _NC), jnp.float8_e4m3fn),
            pltpu.VMEM((_N, 2 * _NC), jnp.bfloat16),
            pltpu.VMEM((_N, 2 * _NC), jnp.float32),
            pltpu.VMEM((2 * _N, 5 * _NC), jnp.bfloat16),
            pltpu.VMEM((_N, _NC), jnp.float32),
            pltpu.VMEM((_N, _NC), jnp.float32),
            pltpu.VMEM((_N, 1), jnp.float32),
            pltpu.VMEM((_N, _NC), jnp.float8_e4m3fn),
            pltpu.VMEM((_N, _NC), jnp.float8_e4m3fn),
            pltpu.SMEM((_NTD,), jnp.float32),
            pltpu.SMEM((_NT,), jnp.float32),
        ],
    )(feat_s, feat_t, W_embed, b_embed.reshape(1, _NC),
      W_tag, b_tag.reshape(1, _NC))
    return res[0, 0]


# speedup vs baseline: 1.3639x; 1.0940x over previous
import jax
import jax.numpy as jnp
from jax.experimental import pallas as pl
from jax.experimental.pallas import tpu as pltpu

_NUM_HOP = 4
_TH = 0.6
_T = 0.07
_NC = 128
_N = 4096
_IN = 256
_TD = 1024
_NTD = _N // _TD
_TILE = 512
_NT = _N // _TILE


def _nt_dot(a, b):
    return jax.lax.dot_general(a, b, (((1,), (1,)), ((), ())),
                               preferred_element_type=jnp.float32)


def _nt_dot_bf(a, b):
    return jax.lax.dot_general(a, b, (((1,), (1,)), ((), ())),
                               preferred_element_type=jnp.float32
                               ).astype(jnp.bfloat16)


def _nn_dot(a, b):
    return jax.lax.dot_general(a, b, (((1,), (0,)), ((), ())),
                               preferred_element_type=jnp.float32)


def _l2norm_rows(x):
    return x / jnp.sqrt(jnp.sum(x * x, axis=1, keepdims=True))


def _gnn_body(feat_s_ref, feat_t_ref, We_ref, be_ref, Wt_ref, bt_ref,
              out_ref,
              F8_ref, Gbf_ref, H_ref, stack_ref, acc_t_ref, acc_s_ref,
              norm_ref, fgt_ref, fgs_ref, cnt_ref, part_ref):
    bf16 = jnp.bfloat16
    fp8 = jnp.float8_e4m3fn
    We = We_ref[...].astype(bf16)
    be = be_ref[...]

    ft = _l2norm_rows(_nn_dot(feat_t_ref[...].astype(bf16), We) + be)
    fs = _l2norm_rows(_nn_dot(feat_s_ref[...].astype(bf16), We) + be)
    H_ref[:, :_NC] = ft
    H_ref[:, _NC:] = fs
    F8_ref[...] = ft.astype(fp8)
    stack_ref[:_N, :_NC] = ft.astype(bf16)
    stack_ref[_N:, :_NC] = fs.astype(bf16)

    def deg_tile(t, carry):
        rows = pl.ds(t * _TD, _TD)
        d = _nt_dot(F8_ref[rows, :], F8_ref[...])
        m = d > _TH
        deg = jnp.sum(m, axis=1, keepdims=True,
                      dtype=jnp.float32) + 1.0
        norm_ref[rows, :] = jax.lax.rsqrt(jnp.maximum(deg, 1.0))
        cnt_ref[t] = jnp.sum(deg) - 2.0 * float(_TD)
        return carry

    jax.lax.fori_loop(0, _NTD, deg_tile, 0)

    tot = cnt_ref[0]
    for t in range(1, _NTD):
        tot = tot + cnt_ref[t]
    bt = bt_ref[...]
    inv_t = 1.0 / _T

    @pl.when(tot > 0.5)
    def _():
        for hop in range(_NUM_HOP):
            Gbf_ref[...] = (H_ref[...] * norm_ref[...]).astype(bf16)

            def hop_tile(t, carry):
                rows = pl.ds(t * _TD, _TD)

                @pl.when(cnt_ref[t] > 0.5)
                def _():
                    d = _nt_dot(F8_ref[rows, :], F8_ref[...])
                    m = (d > _TH).astype(bf16)
                    s = _nn_dot(m, Gbf_ref[...])
                    nrm = norm_ref[rows, :]
                    H_ref[rows, :] = nrm * s + (nrm * nrm) * H_ref[rows, :]

                return carry

            jax.lax.fori_loop(0, _NTD, hop_tile, 0)
            Hb = H_ref[...].astype(bf16)
            c = (hop + 1) * _NC
            stack_ref[:_N, c:c + _NC] = Hb[:, :_NC]
            stack_ref[_N:, c:c + _NC] = Hb[:, _NC:]

        Wt = Wt_ref[...].astype(bf16)
        gt = _l2norm_rows(_nn_dot(stack_ref[:_N, :], Wt) + bt)
        gs = _l2norm_rows(_nn_dot(stack_ref[_N:, :], Wt) + bt)
        acc_t_ref[...] = gt
        acc_s_ref[...] = gs
        fgt_ref[...] = (gt * inv_t).astype(fp8)
        fgs_ref[...] = gs.astype(fp8)

    @pl.when(tot <= 0.5)
    def _():
        Wsum = Wt_ref[0 * _NC:1 * _NC, :]
        for k in range(1, _NUM_HOP + 1):
            Wsum = Wsum + Wt_ref[k * _NC:(k + 1) * _NC, :]
        Wsum = Wsum.astype(bf16)
        gt = _l2norm_rows(_nn_dot(stack_ref[:_N, :_NC], Wsum) + bt)
        gs = _l2norm_rows(_nn_dot(stack_ref[_N:, :_NC], Wsum) + bt)
        acc_t_ref[...] = gt
        acc_s_ref[...] = gs
        fgt_ref[...] = (gt * inv_t).astype(fp8)
        fgs_ref[...] = gs.astype(fp8)

    def nce_tile(t, carry):
        rows = pl.ds(t * _TILE, _TILE)
        logits = _nt_dot(fgt_ref[rows, :], fgs_ref[...])
        e = jnp.exp(logits)
        se_all = jnp.sum(e, axis=1, keepdims=True)
        dlog = jnp.sum(fgt_ref[rows, :].astype(jnp.float32) *
                       fgs_ref[rows, :].astype(jnp.float32), axis=1,
                       keepdims=True)
        pos = jnp.sum(acc_t_ref[rows, :] * acc_s_ref[rows, :], axis=1,
                      keepdims=True) * inv_t
        se = se_all - jnp.exp(dlog) + jnp.exp(pos)
        part_ref[t] = jnp.sum(jnp.log(se) - pos)
        return carry

    jax.lax.fori_loop(0, _NT, nce_tile, 0)

    total = part_ref[0]
    for t in range(1, _NT):
        total = total + part_ref[t]
    out_ref[0, 0] = total * (1.0 / float(_N))


def kernel(feat_s, feat_t, W_embed, b_embed, W_tag, b_tag):
    res = pl.pallas_call(
        _gnn_body,
        out_shape=jax.ShapeDtypeStruct((1, 1), jnp.float32),
        in_specs=[pl.BlockSpec(memory_space=pltpu.VMEM)] * 6,
        out_specs=pl.BlockSpec(memory_space=pltpu.SMEM),
        scratch_shapes=[
            pltpu.VMEM((_N, _NC), jnp.float8_e4m3fn),
            pltpu.VMEM((_N, 2 * _NC), jnp.bfloat16),
            pltpu.VMEM((_N, 2 * _NC), jnp.float32),
            pltpu.VMEM((2 * _N, 5 * _NC), jnp.bfloat16),
            pltpu.VMEM((_N, _NC), jnp.float32),
            pltpu.VMEM((_N, _NC), jnp.float32),
            pltpu.VMEM((_N, 1), jnp.float32),
            pltpu.VMEM((_N, _NC), jnp.float8_e4m3fn),
            pltpu.VMEM((_N, _NC), jnp.float8_e4m3fn),
            pltpu.SMEM((_NTD,), jnp.float32),
            pltpu.SMEM((_NT,), jnp.float32),
        ],
    )(feat_s, feat_t, W_embed, b_embed.reshape(1, _NC),
      W_tag, b_tag.reshape(1, _NC))
    return res[0, 0]
